# Initial kernel scaffold; baseline (speedup 1.0000x reference)
import jax
import jax.numpy as jnp
from jax.experimental import pallas as pl


def kernel(mcc_code, item_id, timestamps, emb_mcc, emb_item):
    raise NotImplementedError("write your pallas kernel here")


# speedup vs baseline: 1.1371x; 1.1371x over previous
import functools

import jax
import jax.numpy as jnp
from jax import lax
from jax.experimental import pallas as pl
from jax.experimental.pallas import tpu as pltpu
from jax.experimental.pallas import tpu_sc as plsc

B = 4096
T = 200
VOCAB = 1000000
EMB = 32
MAX_TIME_DELTA = 86400.0
BT = B * T

NC = 2
NS = 16
NW = NC * NS
ROWS_PER_TILE = BT // NW
BLK = 128
NBLK = ROWS_PER_TILE // BLK
P1_PER_TILE = BT // NS
CHUNK = 3200
NCHUNK_HALF = (P1_PER_TILE // CHUNK) // 2
OUT_W = 2 * EMB + 1


def _rsqrt_scalar(x):
    i = lax.bitcast_convert_type(x, jnp.int32)
    y = lax.bitcast_convert_type(jnp.int32(0x5F3759DF) - (i >> 1),
                                 jnp.float32)
    for _ in range(3):
        y = y * (jnp.float32(1.5) - jnp.float32(0.5) * x * y * y)
    return y


def _body(mcc_hbm, item_hbm, ts_hbm, emb_mcc, emb_item, out_hbm,
          idx_a, idx_b, dbuf, rowbuf, stage, sbuf, shared,
          buf_a, buf_b, b65, sem_a, sem_b):
    c = lax.axis_index("c")
    s = lax.axis_index("s")
    wid = s * NC + c

    lanes = lax.iota(jnp.int32, 16)
    zero16 = jnp.zeros((16,), jnp.float32)

    p1_base = s * P1_PER_TILE

    def delta_vec(i, chunk_off):
        del chunk_off
        pos = i * 16
        cur = rowbuf[pl.ds(16 + pos, 16)]
        prev = rowbuf[pl.ds(15 + pos, 16)]
        d = cur - prev
        at_row_start = jnp.equal(jnp.remainder(pos + lanes, T), 0)
        d = jnp.where(at_row_start, 0.0, d)
        return jnp.minimum(jnp.maximum(d, 0.0), MAX_TIME_DELTA)

    def chunk_pass(k, carry, half, store):
        acc_s, acc_q = carry
        ci = half * NCHUNK_HALF + k
        pltpu.sync_copy(ts_hbm.at[pl.ds(p1_base + ci * CHUNK, CHUNK)],
                        rowbuf.at[pl.ds(16, CHUNK)])

        def inner(i, ic):
            a_s, a_q = ic
            d = delta_vec(i, ci)
            if store:
                dbuf[pl.ds(16 + k * CHUNK + i * 16, 16)] = d
            return a_s + d, a_q + d * d

        return lax.fori_loop(0, CHUNK // 16, inner, (acc_s, acc_q))

    acc = (zero16, zero16)
    acc = lax.fori_loop(
        0, NCHUNK_HALF,
        functools.partial(chunk_pass, half=c, store=True), acc)
    acc = lax.fori_loop(
        0, NCHUNK_HALF,
        functools.partial(chunk_pass, half=1 - c, store=False), acc)
    acc_s, acc_q = acc

    stage[...] = acc_s
    pltpu.sync_copy(stage, shared.at[0, s])
    stage[...] = acc_q
    pltpu.sync_copy(stage, shared.at[1, s])
    plsc.subcore_barrier()
    pltpu.sync_copy(shared, sbuf)

    sum_v = zero16
    sq_v = zero16
    for j in range(NS):
        sum_v = sum_v + sbuf[0, j]
        sq_v = sq_v + sbuf[1, j]
    tot_s = jnp.float32(0.0)
    tot_q = jnp.float32(0.0)
    for j in range(16):
        tot_s = tot_s + sum_v[j]
        tot_q = tot_q + sq_v[j]
    inv_n = jnp.float32(1.0 / BT)
    m_s = tot_s * inv_n
    var_s = tot_q * inv_n - m_s * m_s
    mean = jnp.full((16,), m_s, jnp.float32)
    inv_std = jnp.full((16,), _rsqrt_scalar(var_s + jnp.float32(1e-5)),
                       jnp.float32)

    row0 = wid * ROWS_PER_TILE
    pltpu.sync_copy(mcc_hbm.at[pl.ds(row0, ROWS_PER_TILE)], idx_a)
    pltpu.sync_copy(item_hbm.at[pl.ds(row0, ROWS_PER_TILE)], idx_b)

    lane15 = jnp.equal(lanes, 15)

    def block(j, _):
        base = j * BLK
        cp_a = pltpu.async_copy(emb_mcc.at[idx_a.at[pl.ds(base, BLK)]],
                                buf_a, sem_a)
        cp_b = pltpu.async_copy(emb_item.at[idx_b.at[pl.ds(base, BLK)]],
                                buf_b, sem_b)
        cp_a.wait()
        cp_b.wait()

        def copy_row(r, _c):
            o = r * OUT_W
            b65[pl.ds(o, 16)] = buf_a[r, pl.ds(0, 16)]
            b65[pl.ds(o + 16, 16)] = buf_a[r, pl.ds(16, 16)]
            b65[pl.ds(o + 32, 16)] = buf_b[r, pl.ds(0, 16)]
            b65[pl.ds(o + 48, 16)] = buf_b[r, pl.ds(16, 16)]
            dn = (dbuf[pl.ds(base + r + 1, 16)] - mean) * inv_std
            v = b65[pl.ds(o + 49, 16)]
            b65[pl.ds(o + 49, 16)] = jnp.where(lane15, dn, v)
            return 0

        lax.fori_loop(0, BLK, copy_row, 0)

        pltpu.sync_copy(b65, out_hbm.at[pl.ds((row0 + base) * OUT_W,
                                              BLK * OUT_W)])
        return 0

    lax.fori_loop(0, NBLK, block, 0)


@jax.jit
def _encode(mcc_flat, item_flat, ts_flat, emb_mcc, emb_item):
    mesh = plsc.VectorSubcoreMesh(core_axis_name="c", subcore_axis_name="s",
                                  num_cores=NC, num_subcores=NS)
    f = pl.kernel(
        _body,
        out_type=jax.ShapeDtypeStruct((BT * OUT_W,), jnp.float32),
        mesh=mesh,
        scratch_types=[
            pltpu.VMEM((ROWS_PER_TILE,), jnp.int32),
            pltpu.VMEM((ROWS_PER_TILE,), jnp.int32),
            pltpu.VMEM((ROWS_PER_TILE + 16,), jnp.float32),
            pltpu.VMEM((CHUNK + 16,), jnp.float32),
            pltpu.VMEM((16,), jnp.float32),
            pltpu.VMEM((2, NS, 16), jnp.float32),
            pltpu.VMEM_SHARED((2, NS, 16), jnp.float32),
            pltpu.VMEM((BLK, EMB), jnp.float32),
            pltpu.VMEM((BLK, EMB), jnp.float32),
            pltpu.VMEM((BLK * OUT_W,), jnp.float32),
            pltpu.SemaphoreType.DMA,
            pltpu.SemaphoreType.DMA,
        ],
        compiler_params=pltpu.CompilerParams(use_tc_tiling_on_sc=False),
        name="base_encoder_sc",
    )
    return f(mcc_flat, item_flat, ts_flat, emb_mcc, emb_item)


def kernel(mcc_code, item_id, timestamps, emb_mcc, emb_item):
    out = _encode(mcc_code.reshape(-1), item_id.reshape(-1),
                  timestamps.reshape(-1), emb_mcc, emb_item)
    return out.reshape(B, T, OUT_W)
